# baseline (speedup 1.0000x reference)
import functools
import math

import jax
import jax.numpy as jnp
from jax import lax
from jax.experimental import pallas as pl
from jax.experimental.pallas import tpu as pltpu
from jax.experimental.pallas import tpu_sc as plsc

D_MODEL = 64
SCALE = math.sqrt(D_MODEL)
LANES = 16
NBUF = 4


@functools.lru_cache(maxsize=None)
def _build_call(rows: int, seq: int, vocab: int, d: int):
    info = plsc.get_sparse_core_info()
    nc, ns = info.num_cores, info.num_subcores
    nw = nc * ns
    ipw = rows // nw
    assert ipw == 128 and seq % NBUF == 0 and d == D_MODEL
    mesh = plsc.VectorSubcoreMesh(core_axis_name="c", subcore_axis_name="s")

    @functools.partial(
        pl.kernel,
        mesh=mesh,
        out_type=jax.ShapeDtypeStruct((seq, d // 8, nw, 8, ipw), jnp.float32),
        scratch_types=[pltpu.VMEM((seq, ipw), jnp.int32)]
        + [pltpu.VMEM((2, ipw), jnp.int32) for _ in range(NBUF)]
        + [pltpu.VMEM((ipw, 2 * d), jnp.float32) for _ in range(NBUF)]
        + [pltpu.VMEM((d // 8, 8, ipw), jnp.float32) for _ in range(NBUF)]
        + [pltpu.SemaphoreType.DMA] * (2 * NBUF + 1),
        compiler_params=pltpu.CompilerParams(
            use_tc_tiling_on_sc=True, needs_layout_passes=False
        ),
    )
    def emb(t2_hbm, xt_hbm, out_hbm, xv, *rest):
        meta = rest[:NBUF]
        pair = rest[NBUF : 2 * NBUF]
        outb = rest[2 * NBUF : 3 * NBUF]
        sg = rest[3 * NBUF : 4 * NBUF]
        sw = rest[4 * NBUF : 5 * NBUF]
        sx = rest[5 * NBUF]
        wid = lax.axis_index("s") * nc + lax.axis_index("c")

        @pl.loop(0, seq)
        def stage(j):
            pltpu.async_copy(xt_hbm.at[j * nw + wid], xv.at[j], sx)

        @pl.loop(0, seq)
        def stage_wait(j):
            pltpu.make_async_copy(xt_hbm.at[wid], xv.at[0], sx).wait()

        def prep_and_gather(j, b):
            for tg in range(ipw // LANES):
                sl = pl.ds(tg * LANES, LANES)
                iv = xv[j, sl]
                meta[b][0, sl] = iv >> 1
                meta[b][1, sl] = (iv & 1) << 6
            pltpu.async_copy(t2_hbm.at[meta[b].at[0]], pair[b], sg[b])

        def out_slice(j):
            return out_hbm.at[j, pl.ds(0, d // 8), wid]

        for b in range(NBUF - 1):
            prep_and_gather(b, b)

        @pl.loop(0, seq, step=NBUF)
        def outer(jj):
            for b in range(NBUF):
                j = jj + b
                pltpu.make_async_copy(
                    t2_hbm.at[meta[b].at[0]], pair[b], sg[b]
                ).wait()

                for tg in range(ipw // LANES):
                    sl = pl.ds(tg * LANES, LANES)
                    tvec = lax.iota(jnp.int32, LANES) + (tg * LANES)
                    a64 = meta[b][1, sl]

                    @pl.loop(0, d // 8)
                    def kt_loop(kt, tvec=tvec, a64=a64, sl=sl, b=b):
                        for ks in range(8):
                            v = plsc.load_gather(
                                pair[b], [tvec, a64 + (kt * 8 + ks)]
                            )
                            outb[b][kt, ks, sl] = v * SCALE

                pltpu.async_copy(outb[b], out_slice(j), sw[b])

                nxt = j + NBUF - 1
                bf = (b + NBUF - 1) % NBUF

                @pl.when(jnp.logical_and(nxt < seq, j >= 1))
                def _():
                    pltpu.make_async_copy(outb[bf], out_slice(0), sw[bf]).wait()
                    prep_and_gather(nxt, bf)

                @pl.when(jnp.logical_and(nxt < seq, j < 1))
                def _():
                    prep_and_gather(nxt, bf)

        for b in range(NBUF):
            pltpu.make_async_copy(outb[b], out_slice(0), sw[b]).wait()

    return emb


def kernel(x, table):
    vocab, d = table.shape
    rows, seq = x.shape
    t2 = jnp.reshape(table, (vocab // 2, 2 * d))
    xt = jnp.reshape(jnp.swapaxes(x.astype(jnp.int32), 0, 1), (-1, 128))
    o5 = _build_call(rows, seq, vocab, d)(t2, xt)
    return jnp.transpose(o5, (2, 4, 0, 1, 3)).reshape(rows, seq, d)


# speedup vs baseline: 1.6037x; 1.6037x over previous
import functools
import math

import jax
import jax.numpy as jnp
from jax import lax
from jax.experimental import pallas as pl
from jax.experimental.pallas import tpu as pltpu
from jax.experimental.pallas import tpu_sc as plsc

D_MODEL = 64
SCALE = math.sqrt(D_MODEL)
LANES = 16
NBUF = 4


@functools.lru_cache(maxsize=None)
def _build_call(rows: int, seq: int, vocab: int, d: int):
    info = plsc.get_sparse_core_info()
    nc, ns = info.num_cores, info.num_subcores
    nw = nc * ns
    assert rows % (nw * NBUF) == 0 and seq % 8 == 0
    r_per_w = rows // nw
    mesh = plsc.VectorSubcoreMesh(core_axis_name="c", subcore_axis_name="s")

    @functools.partial(
        pl.kernel,
        mesh=mesh,
        out_type=jax.ShapeDtypeStruct((rows, seq, d), jnp.float32),
        scratch_types=[
            pltpu.VMEM((r_per_w, seq), jnp.int32),
            pltpu.VMEM((NBUF, seq, d), jnp.float32),
        ]
        + [pltpu.SemaphoreType.DMA] * (2 * NBUF),
        compiler_params=pltpu.CompilerParams(use_tc_tiling_on_sc=False),
    )
    def emb(table_hbm, x_hbm, out_hbm, idx_v, rows_v, *sems):
        sg, sw = sems[:NBUF], sems[NBUF:]
        wid = lax.axis_index("s") * nc + lax.axis_index("c")
        base = wid * r_per_w
        pltpu.sync_copy(x_hbm.at[pl.ds(base, r_per_w)], idx_v)

        def gather_start(g, b):
            pltpu.async_copy(table_hbm.at[idx_v.at[g]], rows_v.at[b], sg[b])

        for b in range(NBUF - 1):
            gather_start(b, b)

        @pl.loop(0, r_per_w, step=NBUF)
        def outer(gg):
            for b in range(NBUF):
                g = gg + b
                pltpu.make_async_copy(
                    table_hbm.at[idx_v.at[0]], rows_v.at[b], sg[b]
                ).wait()

                @plsc.parallel_loop(0, seq, unroll=4)
                def mul(i):
                    for j in range(d // LANES):
                        sl = pl.ds(j * LANES, LANES)
                        rows_v[b, i, sl] = rows_v[b, i, sl] * SCALE

                pltpu.async_copy(rows_v.at[b], out_hbm.at[base + g], sw[b])

                nxt = g + NBUF - 1
                bf = (b + NBUF - 1) % NBUF

                @pl.when(jnp.logical_and(nxt < r_per_w, g >= 1))
                def _():
                    pltpu.make_async_copy(
                        rows_v.at[bf], out_hbm.at[base], sw[bf]
                    ).wait()
                    gather_start(nxt, bf)

                @pl.when(jnp.logical_and(nxt < r_per_w, g < 1))
                def _():
                    gather_start(nxt, bf)

        for b in range(NBUF):
            pltpu.make_async_copy(rows_v.at[b], out_hbm.at[base], sw[b]).wait()

    return emb


def kernel(x, table):
    vocab, d = table.shape
    rows, seq = x.shape
    return _build_call(rows, seq, vocab, d)(table, x.astype(jnp.int32))
